# baseline (speedup 1.0000x reference)
import functools

import jax
import jax.numpy as jnp
import numpy as np
from jax import lax
from jax.experimental import pallas as pl
from jax.experimental.pallas import tpu as pltpu
from jax.experimental.pallas import tpu_sc as plsc

N = 10000
D = 128
K = 32

NW = 32
NP = 10240
C = NP // NW
NB = 4
NBUF = 2
NCHUNK = C // NB
VPR = D // 16


def _sc_gather_mean_body(x_hbm, idx_hbm, out_hbm, x_spmem, idx_v, rows0,
                         rows1, outb0, outb1, sg0, sg1, so0, so1):
    wid = lax.axis_index("s") * 2 + lax.axis_index("c")
    node_base = wid * C
    nchunk_w = jnp.minimum(NCHUNK, (N - node_base) // NB)
    stage_base = jnp.minimum(node_base * K, N * K - C * K)
    idx_off = node_base * K - stage_base
    pltpu.sync_copy(idx_hbm.at[pl.ds(stage_base, C * K)], idx_v)

    sid = lax.axis_index("s")
    rpt = 624
    pltpu.sync_copy(x_hbm.at[pl.ds(sid * rpt, rpt)],
                    x_spmem.at[pl.ds(sid * rpt, rpt)])
    @pl.when(sid == 0)
    def _():
        pltpu.sync_copy(x_hbm.at[pl.ds(16 * rpt, N - 16 * rpt)],
                        x_spmem.at[pl.ds(16 * rpt, N - 16 * rpt)])
    plsc.subcore_barrier()

    rows = (rows0, rows1)
    outb = (outb0, outb1)
    sg = (sg0, sg1)
    so = (so0, so1)

    def gather(gi, b):
        off = jnp.minimum(idx_off + gi * (NB * K), C * K - NB * K)
        return pltpu.make_async_copy(
            x_spmem.at[idx_v.at[pl.ds(off, NB * K)]], rows[b], sg[b])

    def outcopy(gi, b):
        row = jnp.minimum(node_base + gi * NB, N - NB)
        return pltpu.make_async_copy(
            outb[b], out_hbm.at[pl.ds(row, NB)], so[b])

    for b in range(NBUF):
        gather(b, b).start()

    @pl.loop(0, NCHUNK, step=NBUF)
    def _chunks(g):
        for b in range(NBUF):
            gi = g + b
            gather(gi, b).wait()
            @pl.when((gi >= NBUF) & (gi < nchunk_w))
            def _():
                outcopy(gi, b).wait()
            @pl.loop(0, NB)
            def _node(j):
                accs = [jnp.zeros((16,), jnp.float32) for _ in range(VPR)]
                for k in range(K):
                    for q in range(VPR // 2):
                        v = plsc.bitcast(
                            rows[b][j * K + k, pl.ds(q * 16, 16)], jnp.int32)
                        ve = plsc.bitcast(v << 16, jnp.float32)
                        vo = plsc.bitcast(v & jnp.int32(-65536), jnp.float32)
                        accs[2 * q] = accs[2 * q] + ve
                        accs[2 * q + 1] = accs[2 * q + 1] + vo
                for q in range(VPR // 2):
                    outb[b][j, pl.ds(q * 32, 16)] = accs[2 * q] * (1.0 / K)
                    outb[b][j, pl.ds(q * 32 + 16, 16)] = \
                        accs[2 * q + 1] * (1.0 / K)
            @pl.when(gi + NBUF < NCHUNK)
            def _():
                gather(gi + NBUF, b).start()
            @pl.when(gi < nchunk_w)
            def _():
                outcopy(gi, b).start()

    for b in range(NBUF):
        outcopy(nchunk_w - NBUF + b, b).wait()


@functools.partial(jax.jit, static_argnames=())
def _sc_gather_mean(x, idx_flat):
    kern = pl.kernel(
        _sc_gather_mean_body,
        out_type=jax.ShapeDtypeStruct((N, D), jnp.float32),
        mesh=plsc.VectorSubcoreMesh(core_axis_name="c", subcore_axis_name="s"),
        compiler_params=pltpu.CompilerParams(needs_layout_passes=False),
        scratch_types=[
            pltpu.VMEM_SHARED((N, D), jnp.float32),
            pltpu.VMEM((C * K,), jnp.int32),
            pltpu.VMEM((NB * K, D), jnp.float32),
            pltpu.VMEM((NB * K, D), jnp.float32),
            pltpu.VMEM((NB, D), jnp.float32),
            pltpu.VMEM((NB, D), jnp.float32),
            pltpu.SemaphoreType.DMA,
            pltpu.SemaphoreType.DMA,
            pltpu.SemaphoreType.DMA,
            pltpu.SemaphoreType.DMA,
        ],
    )
    return kern(x, idx_flat)


def _linear_body(x_ref, a_ref, w_ref, b_ref, o_ref, *, act):
    wa = w_ref[:, :D]
    wb = w_ref[:, D:]
    acc = lax.dot_general(x_ref[...], wa, (((1,), (1,)), ((), ())),
                          preferred_element_type=jnp.float32)
    acc = acc + lax.dot_general(a_ref[...], wb, (((1,), (1,)), ((), ())),
                                preferred_element_type=jnp.float32)
    acc = acc + b_ref[...]
    if act:
        acc = jnp.maximum(acc, 0.0)
    o_ref[...] = acc


def _linear(x, agg, w, b, act):
    BM = 1000
    grid = (N // BM,)
    return pl.pallas_call(
        functools.partial(_linear_body, act=act),
        out_shape=jax.ShapeDtypeStruct((N, D), jnp.float32),
        grid=grid,
        in_specs=[
            pl.BlockSpec((BM, D), lambda i: (i, 0)),
            pl.BlockSpec((BM, D), lambda i: (i, 0)),
            pl.BlockSpec((D, 2 * D), lambda i: (0, 0)),
            pl.BlockSpec((1, D), lambda i: (0, 0)),
        ],
        out_specs=pl.BlockSpec((BM, D), lambda i: (i, 0)),
    )(x, agg, w, b)


_UNPACK_PERM = np.arange(D).reshape(4, 16, 2).transpose(0, 2, 1).reshape(D)


def kernel(node_features, neigh_idx, W1, b1, W2, b2):
    idx = neigh_idx.astype(jnp.int32).reshape(N * K)
    b1r = b1.reshape(1, D)
    b2r = b2.reshape(1, D)

    w1m = jnp.concatenate([W1[:, :D], W1[:, D + _UNPACK_PERM]], axis=1)
    w2m = jnp.concatenate([W2[:, :D], W2[:, D + _UNPACK_PERM]], axis=1)

    def pack32(v):
        vb = v.astype(jnp.bfloat16).reshape(N, D // 2, 2)
        packed = lax.bitcast_convert_type(
            lax.bitcast_convert_type(vb, jnp.int32), jnp.float32)
        return jnp.concatenate(
            [packed, jnp.zeros((N, D // 2), jnp.float32)], axis=1)

    agg1 = _sc_gather_mean(pack32(node_features), idx)
    h = _linear(node_features, agg1, w1m, b1r, act=True)
    agg2 = _sc_gather_mean(pack32(h), idx)
    out = _linear(h, agg2, w2m, b2r, act=False)
    return out


# speedup vs baseline: 1.0220x; 1.0220x over previous
import functools

import jax
import jax.numpy as jnp
from jax import lax
from jax.experimental import pallas as pl
from jax.experimental.pallas import tpu as pltpu
from jax.experimental.pallas import tpu_sc as plsc

N = 10000
D = 128
K = 32

NW = 32
NP = 10240
C = NP // NW
NB = 4
NBUF = 2
NCHUNK = C // NB
VPR = D // 16


def _sc_gather_mean_body(x_hbm, idx_hbm, out_hbm, x_spmem, idx_v, rows0,
                         rows1, outb0, outb1, sg0, sg1, so0, so1):
    wid = lax.axis_index("s") * 2 + lax.axis_index("c")
    node_base = wid * C
    nchunk_w = jnp.minimum(NCHUNK, (N - node_base) // NB)
    stage_base = jnp.minimum(node_base * K, N * K - C * K)
    idx_off = node_base * K - stage_base
    pltpu.sync_copy(idx_hbm.at[pl.ds(stage_base, C * K)], idx_v)

    sid = lax.axis_index("s")
    rpt = 624
    pltpu.sync_copy(x_hbm.at[pl.ds(sid * rpt, rpt)],
                    x_spmem.at[pl.ds(sid * rpt, rpt)])
    @pl.when(sid == 0)
    def _():
        pltpu.sync_copy(x_hbm.at[pl.ds(16 * rpt, N - 16 * rpt)],
                        x_spmem.at[pl.ds(16 * rpt, N - 16 * rpt)])
    plsc.subcore_barrier()

    rows = (rows0, rows1)
    outb = (outb0, outb1)
    sg = (sg0, sg1)
    so = (so0, so1)

    def gather(gi, b):
        off = jnp.minimum(idx_off + gi * (NB * K), C * K - NB * K)
        return pltpu.make_async_copy(
            x_spmem.at[idx_v.at[pl.ds(off, NB * K)]], rows[b], sg[b])

    def outcopy(gi, b):
        row = jnp.minimum(node_base + gi * NB, N - NB)
        return pltpu.make_async_copy(
            outb[b], out_hbm.at[pl.ds(row, NB)], so[b])

    for b in range(NBUF):
        gather(b, b).start()

    @pl.loop(0, NCHUNK, step=NBUF)
    def _chunks(g):
        for b in range(NBUF):
            gi = g + b
            gather(gi, b).wait()
            @pl.when((gi >= NBUF) & (gi < nchunk_w))
            def _():
                outcopy(gi, b).wait()
            @pl.loop(0, NB)
            def _node(j):
                accs = [jnp.zeros((16,), jnp.float32) for _ in range(VPR)]
                for k in range(K):
                    for dd in range(VPR):
                        accs[dd] = accs[dd] + rows[b][j * K + k,
                                                      pl.ds(dd * 16, 16)]
                for dd in range(VPR):
                    outb[b][j, pl.ds(dd * 16, 16)] = accs[dd] * (1.0 / K)
            @pl.when(gi + NBUF < NCHUNK)
            def _():
                gather(gi + NBUF, b).start()
            @pl.when(gi < nchunk_w)
            def _():
                outcopy(gi, b).start()

    for b in range(NBUF):
        outcopy(nchunk_w - NBUF + b, b).wait()


@functools.partial(jax.jit, static_argnames=())
def _sc_gather_mean(x, idx_flat):
    kern = pl.kernel(
        _sc_gather_mean_body,
        out_type=jax.ShapeDtypeStruct((N, D), jnp.float32),
        mesh=plsc.VectorSubcoreMesh(core_axis_name="c", subcore_axis_name="s"),
        scratch_types=[
            pltpu.VMEM_SHARED((N, D), jnp.float32),
            pltpu.VMEM((C * K,), jnp.int32),
            pltpu.VMEM((NB * K, D), jnp.float32),
            pltpu.VMEM((NB * K, D), jnp.float32),
            pltpu.VMEM((NB, D), jnp.float32),
            pltpu.VMEM((NB, D), jnp.float32),
            pltpu.SemaphoreType.DMA,
            pltpu.SemaphoreType.DMA,
            pltpu.SemaphoreType.DMA,
            pltpu.SemaphoreType.DMA,
        ],
    )
    return kern(x, idx_flat)


def _self_body(x_ref, w_ref, b_ref, o_ref):
    o_ref[...] = lax.dot_general(
        x_ref[...], w_ref[:, :D], (((1,), (1,)), ((), ())),
        preferred_element_type=jnp.float32) + b_ref[...]


def _comb_body(s_ref, a_ref, w_ref, o_ref, *, act):
    acc = s_ref[...] + lax.dot_general(
        a_ref[...], w_ref[:, D:], (((1,), (1,)), ((), ())),
        preferred_element_type=jnp.float32)
    if act:
        acc = jnp.maximum(acc, 0.0)
    o_ref[...] = acc


_BM = 1000


def _linear_self(x, w, b):
    return pl.pallas_call(
        _self_body,
        out_shape=jax.ShapeDtypeStruct((N, D), jnp.float32),
        grid=(N // _BM,),
        in_specs=[
            pl.BlockSpec((_BM, D), lambda i: (i, 0)),
            pl.BlockSpec((D, 2 * D), lambda i: (0, 0)),
            pl.BlockSpec((1, D), lambda i: (0, 0)),
        ],
        out_specs=pl.BlockSpec((_BM, D), lambda i: (i, 0)),
    )(x, w, b)


def _linear_comb(s, agg, w, act):
    return pl.pallas_call(
        functools.partial(_comb_body, act=act),
        out_shape=jax.ShapeDtypeStruct((N, D), jnp.float32),
        grid=(N // _BM,),
        in_specs=[
            pl.BlockSpec((_BM, D), lambda i: (i, 0)),
            pl.BlockSpec((_BM, D), lambda i: (i, 0)),
            pl.BlockSpec((D, 2 * D), lambda i: (0, 0)),
        ],
        out_specs=pl.BlockSpec((_BM, D), lambda i: (i, 0)),
    )(s, agg, w)


def kernel(node_features, neigh_idx, W1, b1, W2, b2):
    idx = neigh_idx.astype(jnp.int32).reshape(N * K)
    b1r = b1.reshape(1, D)
    b2r = b2.reshape(1, D)

    agg1 = _sc_gather_mean(node_features, idx)
    s1 = _linear_self(node_features, W1, b1r)
    h = _linear_comb(s1, agg1, W1, act=True)

    agg2 = _sc_gather_mean(h, idx)
    s2 = _linear_self(h, W2, b2r)
    out = _linear_comb(s2, agg2, W2, act=False)
    return out


# speedup vs baseline: 1.0422x; 1.0198x over previous
import functools

import jax
import jax.numpy as jnp
from jax import lax
from jax.experimental import pallas as pl
from jax.experimental.pallas import tpu as pltpu
from jax.experimental.pallas import tpu_sc as plsc

N = 10000
D = 128
K = 32

NW = 32
NP = 10240
C = NP // NW
NB = 4
NBUF = 2
NCHUNK = C // NB
VPR = D // 16


def _sc_gather_mean_body(x_hbm, idx_hbm, out_hbm, x_spmem, idx_v, rows0,
                         rows1, outb0, outb1, sg0, sg1, so0, so1):
    wid = lax.axis_index("s") * 2 + lax.axis_index("c")
    node_base = wid * C
    nchunk_w = jnp.minimum(NCHUNK, (N - node_base) // NB)
    stage_base = jnp.minimum(node_base * K, N * K - C * K)
    idx_off = node_base * K - stage_base
    pltpu.sync_copy(idx_hbm.at[pl.ds(stage_base, C * K)], idx_v)

    sid = lax.axis_index("s")
    rpt = 624
    pltpu.sync_copy(x_hbm.at[pl.ds(sid * rpt, rpt)],
                    x_spmem.at[pl.ds(sid * rpt, rpt)])
    @pl.when(sid == 0)
    def _():
        pltpu.sync_copy(x_hbm.at[pl.ds(16 * rpt, N - 16 * rpt)],
                        x_spmem.at[pl.ds(16 * rpt, N - 16 * rpt)])
    plsc.subcore_barrier()

    rows = (rows0, rows1)
    outb = (outb0, outb1)
    sg = (sg0, sg1)
    so = (so0, so1)

    def gather(gi, b):
        off = jnp.minimum(idx_off + gi * (NB * K), C * K - NB * K)
        return pltpu.make_async_copy(
            x_spmem.at[idx_v.at[pl.ds(off, NB * K)]], rows[b], sg[b])

    def outcopy(gi, b):
        row = jnp.minimum(node_base + gi * NB, N - NB)
        return pltpu.make_async_copy(
            outb[b], out_hbm.at[pl.ds(row, NB)], so[b])

    for b in range(NBUF):
        gather(b, b).start()

    @pl.loop(0, NCHUNK, step=NBUF)
    def _chunks(g):
        for b in range(NBUF):
            gi = g + b
            gather(gi, b).wait()
            @pl.when((gi >= NBUF) & (gi < nchunk_w))
            def _():
                outcopy(gi, b).wait()
            @pl.loop(0, NB)
            def _node(j):
                accs = [jnp.zeros((16,), jnp.float32) for _ in range(VPR)]
                for k in range(K):
                    for dd in range(VPR):
                        accs[dd] = accs[dd] + rows[b][j * K + k,
                                                      pl.ds(dd * 16, 16)]
                for dd in range(VPR):
                    outb[b][j, pl.ds(dd * 16, 16)] = accs[dd] * (1.0 / K)
            @pl.when(gi + NBUF < NCHUNK)
            def _():
                gather(gi + NBUF, b).start()
            @pl.when(gi < nchunk_w)
            def _():
                outcopy(gi, b).start()

    for b in range(NBUF):
        outcopy(nchunk_w - NBUF + b, b).wait()


@functools.partial(jax.jit, static_argnames=())
def _sc_gather_mean(x, idx_flat):
    kern = pl.kernel(
        _sc_gather_mean_body,
        out_type=jax.ShapeDtypeStruct((N, D), jnp.float32),
        mesh=plsc.VectorSubcoreMesh(core_axis_name="c", subcore_axis_name="s"),
        scratch_types=[
            pltpu.VMEM_SHARED((N, D), jnp.float32),
            pltpu.VMEM((C * K,), jnp.int32),
            pltpu.VMEM((NB * K, D), jnp.float32),
            pltpu.VMEM((NB * K, D), jnp.float32),
            pltpu.VMEM((NB, D), jnp.float32),
            pltpu.VMEM((NB, D), jnp.float32),
            pltpu.SemaphoreType.DMA,
            pltpu.SemaphoreType.DMA,
            pltpu.SemaphoreType.DMA,
            pltpu.SemaphoreType.DMA,
        ],
    )
    return kern(x, idx_flat)


def _self_body(x_ref, w_ref, b_ref, o_ref):
    o_ref[...] = lax.dot_general(
        x_ref[...], w_ref[:, :D], (((1,), (1,)), ((), ())),
        preferred_element_type=jnp.float32) + b_ref[...]


def _comb_body(s_ref, a_ref, w_ref, o_ref, *, act):
    acc = s_ref[...] + lax.dot_general(
        a_ref[...], w_ref[:, D:], (((1,), (1,)), ((), ())),
        preferred_element_type=jnp.float32)
    if act:
        acc = jnp.maximum(acc, 0.0)
    o_ref[...] = acc


_BM = 2000


def _linear_self(x, w, b):
    return pl.pallas_call(
        _self_body,
        out_shape=jax.ShapeDtypeStruct((N, D), jnp.float32),
        grid=(N // _BM,),
        in_specs=[
            pl.BlockSpec((_BM, D), lambda i: (i, 0)),
            pl.BlockSpec((D, 2 * D), lambda i: (0, 0)),
            pl.BlockSpec((1, D), lambda i: (0, 0)),
        ],
        out_specs=pl.BlockSpec((_BM, D), lambda i: (i, 0)),
    )(x, w, b)


def _linear_comb(s, agg, w, act):
    return pl.pallas_call(
        functools.partial(_comb_body, act=act),
        out_shape=jax.ShapeDtypeStruct((N, D), jnp.float32),
        grid=(N // _BM,),
        in_specs=[
            pl.BlockSpec((_BM, D), lambda i: (i, 0)),
            pl.BlockSpec((_BM, D), lambda i: (i, 0)),
            pl.BlockSpec((D, 2 * D), lambda i: (0, 0)),
        ],
        out_specs=pl.BlockSpec((_BM, D), lambda i: (i, 0)),
    )(s, agg, w)


def kernel(node_features, neigh_idx, W1, b1, W2, b2):
    idx = neigh_idx.astype(jnp.int32).reshape(N * K)
    b1r = b1.reshape(1, D)
    b2r = b2.reshape(1, D)

    agg1 = _sc_gather_mean(node_features, idx)
    s1 = _linear_self(node_features, W1, b1r)
    h = _linear_comb(s1, agg1, W1, act=True)

    agg2 = _sc_gather_mean(h, idx)
    s2 = _linear_self(h, W2, b2r)
    out = _linear_comb(s2, agg2, W2, act=False)
    return out


# speedup vs baseline: 1.0638x; 1.0207x over previous
import functools

import jax
import jax.numpy as jnp
from jax import lax
from jax.experimental import pallas as pl
from jax.experimental.pallas import tpu as pltpu
from jax.experimental.pallas import tpu_sc as plsc

N = 10000
D = 128
K = 32

NW = 32
NP = 10240
C = NP // NW
NB = 4
NBUF = 2
NCHUNK = C // NB
VPR = D // 16


def _sc_gather_mean_body(x_hbm, idx_hbm, out_hbm, x_spmem, idx_v, rows0,
                         rows1, outb0, outb1, sg0, sg1, so0, so1):
    wid = lax.axis_index("s") * 2 + lax.axis_index("c")
    node_base = wid * C
    nchunk_w = jnp.minimum(NCHUNK, (N - node_base) // NB)
    stage_base = jnp.minimum(node_base * K, N * K - C * K)
    idx_off = node_base * K - stage_base
    pltpu.sync_copy(idx_hbm.at[pl.ds(stage_base, C * K)], idx_v)

    sid = lax.axis_index("s")
    rpt = 624
    pltpu.sync_copy(x_hbm.at[pl.ds(sid * rpt, rpt)],
                    x_spmem.at[pl.ds(sid * rpt, rpt)])
    @pl.when(sid == 0)
    def _():
        pltpu.sync_copy(x_hbm.at[pl.ds(16 * rpt, N - 16 * rpt)],
                        x_spmem.at[pl.ds(16 * rpt, N - 16 * rpt)])
    plsc.subcore_barrier()

    rows = (rows0, rows1)
    outb = (outb0, outb1)
    sg = (sg0, sg1)
    so = (so0, so1)

    def gather(gi, b):
        off = jnp.minimum(idx_off + gi * (NB * K), C * K - NB * K)
        return pltpu.make_async_copy(
            x_spmem.at[idx_v.at[pl.ds(off, NB * K)]], rows[b], sg[b])

    def outcopy(gi, b):
        row = jnp.minimum(node_base + gi * NB, N - NB)
        return pltpu.make_async_copy(
            outb[b], out_hbm.at[pl.ds(row, NB)], so[b])

    for b in range(NBUF):
        gather(b, b).start()

    @pl.loop(0, NCHUNK, step=NBUF)
    def _chunks(g):
        for b in range(NBUF):
            gi = g + b
            gather(gi, b).wait()
            @pl.when((gi >= NBUF) & (gi < nchunk_w))
            def _():
                outcopy(gi, b).wait()
            @pl.loop(0, NB)
            def _node(j):
                accs = [jnp.zeros((16,), jnp.float32) for _ in range(VPR)]
                for k in range(K):
                    for dd in range(VPR):
                        accs[dd] = accs[dd] + rows[b][j * K + k,
                                                      pl.ds(dd * 16, 16)]
                for dd in range(VPR):
                    outb[b][j, pl.ds(dd * 16, 16)] = accs[dd] * (1.0 / K)
            @pl.when(gi + NBUF < NCHUNK)
            def _():
                gather(gi + NBUF, b).start()
            @pl.when(gi < nchunk_w)
            def _():
                outcopy(gi, b).start()

    for b in range(NBUF):
        outcopy(nchunk_w - NBUF + b, b).wait()


@functools.partial(jax.jit, static_argnames=())
def _sc_gather_mean(x, idx_flat):
    kern = pl.kernel(
        _sc_gather_mean_body,
        out_type=jax.ShapeDtypeStruct((N, D), jnp.float32),
        mesh=plsc.VectorSubcoreMesh(core_axis_name="c", subcore_axis_name="s"),
        scratch_types=[
            pltpu.VMEM_SHARED((N, D), jnp.float32),
            pltpu.VMEM((C * K,), jnp.int32),
            pltpu.VMEM((NB * K, D), jnp.float32),
            pltpu.VMEM((NB * K, D), jnp.float32),
            pltpu.VMEM((NB, D), jnp.float32),
            pltpu.VMEM((NB, D), jnp.float32),
            pltpu.SemaphoreType.DMA,
            pltpu.SemaphoreType.DMA,
            pltpu.SemaphoreType.DMA,
            pltpu.SemaphoreType.DMA,
        ],
    )
    return kern(x, idx_flat)


def _self_body(x_ref, w_ref, b_ref, o_ref):
    o_ref[...] = lax.dot_general(
        x_ref[...], w_ref[:, :D], (((1,), (1,)), ((), ())),
        preferred_element_type=jnp.float32) + b_ref[...]


def _comb_body(s_ref, a_ref, w_ref, o_ref, *, act):
    acc = s_ref[...] + lax.dot_general(
        a_ref[...], w_ref[:, D:], (((1,), (1,)), ((), ())),
        preferred_element_type=jnp.float32)
    if act:
        acc = jnp.maximum(acc, 0.0)
    o_ref[...] = acc


_BM = 5000


def _linear_self(x, w, b):
    return pl.pallas_call(
        _self_body,
        out_shape=jax.ShapeDtypeStruct((N, D), jnp.float32),
        grid=(N // _BM,),
        in_specs=[
            pl.BlockSpec((_BM, D), lambda i: (i, 0)),
            pl.BlockSpec((D, 2 * D), lambda i: (0, 0)),
            pl.BlockSpec((1, D), lambda i: (0, 0)),
        ],
        out_specs=pl.BlockSpec((_BM, D), lambda i: (i, 0)),
    )(x, w, b)


def _linear_comb(s, agg, w, act):
    return pl.pallas_call(
        functools.partial(_comb_body, act=act),
        out_shape=jax.ShapeDtypeStruct((N, D), jnp.float32),
        grid=(N // _BM,),
        in_specs=[
            pl.BlockSpec((_BM, D), lambda i: (i, 0)),
            pl.BlockSpec((_BM, D), lambda i: (i, 0)),
            pl.BlockSpec((D, 2 * D), lambda i: (0, 0)),
        ],
        out_specs=pl.BlockSpec((_BM, D), lambda i: (i, 0)),
    )(s, agg, w)


def kernel(node_features, neigh_idx, W1, b1, W2, b2):
    idx = neigh_idx.astype(jnp.int32).reshape(N * K)
    b1r = b1.reshape(1, D)
    b2r = b2.reshape(1, D)

    agg1 = _sc_gather_mean(node_features, idx)
    s1 = _linear_self(node_features, W1, b1r)
    h = _linear_comb(s1, agg1, W1, act=True)

    agg2 = _sc_gather_mean(h, idx)
    s2 = _linear_self(h, W2, b2r)
    out = _linear_comb(s2, agg2, W2, act=False)
    return out
